# baseline (speedup 1.0000x reference)
import jax
import jax.numpy as jnp
from jax import lax
from jax.experimental import pallas as pl
from jax.experimental.pallas import tpu as pltpu
from jax.experimental.pallas import tpu_sc as plsc

_L = 16
_DIM = 64


def _scs_body(idx_hbm, a_hbm, b_hbm, c1_hbm, c2_hbm, out_hbm,
              idx_s, rowa_s, rowb_s, c_s, res_s, sem):
    @pl.when(lax.axis_index("c") == 0)
    def _():
        pltpu.sync_copy(idx_hbm, idx_s)
        xs = idx_s[0]
        ys = idx_s[8]
        da = pltpu.async_copy(a_hbm.at[pl.ds(xs, 1)], rowa_s, sem)
        db = pltpu.async_copy(b_hbm.at[pl.ds(ys, 1)], rowb_s, sem)
        dc = pltpu.async_copy(c1_hbm.at[pl.ds(xs, 1), :], c_s.at[pl.ds(0, 1), :], sem)
        dd = pltpu.async_copy(c2_hbm.at[pl.ds(ys, 1), :], c_s.at[pl.ds(1, 1), :], sem)
        da.wait()
        db.wait()
        dc.wait()
        dd.wait()
        acc = c_s[0, 0] + c_s[1, 0]
        for i in range(_DIM):
            acc = acc + rowa_s[0, i] * rowb_s[0, i]
        res_s[0] = acc
        pltpu.sync_copy(res_s, out_hbm)


def kernel(x, y, A, B, c1, c2):
    idx = jnp.concatenate([
        jnp.full((8,), x, dtype=jnp.int32),
        jnp.full((8,), y, dtype=jnp.int32),
    ])
    run = pl.kernel(
        _scs_body,
        mesh=plsc.ScalarSubcoreMesh(axis_name="c"),
        out_type=jax.ShapeDtypeStruct((_L,), jnp.float32),
        scratch_types=[
            pltpu.SMEM((_L,), jnp.int32),
            pltpu.SMEM((1, _DIM), jnp.float32),
            pltpu.SMEM((1, _DIM), jnp.float32),
            pltpu.SMEM((2, 1), jnp.float32),
            pltpu.SMEM((_L,), jnp.float32),
            pltpu.SemaphoreType.DMA,
        ],
    )
    out = run(idx, A, B, c1, c2)
    return out[:1]


# speedup vs baseline: 5.2085x; 5.2085x over previous
import jax
import jax.numpy as jnp
from jax import lax
from jax.experimental import pallas as pl
from jax.experimental.pallas import tpu as pltpu
from jax.experimental.pallas import tpu_sc as plsc

_L = 16
_DIM = 64
_NROW = 100000


def _tec_body(idx_hbm, at_hbm, bt_hbm, c1_hbm, c2_hbm, out_hbm,
              ixv, ab0, ab1, ab2, ab3, ab4, ab5, ab6, ab7,
              bb0, bb1, bb2, bb3, bb4, bb5, bb6, bb7,
              cb1, cb2, res, sem):
    abufs = [ab0, ab1, ab2, ab3, ab4, ab5, ab6, ab7]
    bbufs = [bb0, bb1, bb2, bb3, bb4, bb5, bb6, bb7]
    cid = lax.axis_index("c")
    sid = lax.axis_index("s")

    @pl.when(jnp.logical_and(cid == 0, sid == 0))
    def _():
        pltpu.sync_copy(idx_hbm, ixv)
        iv = ixv[...]
        xs = iv[0]
        ys = iv[8]
        xt = pl.multiple_of((xs // 128) * 128, 128)
        yt = pl.multiple_of((ys // 128) * 128, 128)
        cps = []
        for k in range(8):
            cps.append(pltpu.async_copy(
                at_hbm.at[pl.ds(8 * k, 8), pl.ds(xt, 128)], abufs[k], sem))
            cps.append(pltpu.async_copy(
                bt_hbm.at[pl.ds(8 * k, 8), pl.ds(yt, 128)], bbufs[k], sem))
        cps.append(pltpu.async_copy(c1_hbm.at[pl.ds(xt, 128)], cb1, sem))
        cps.append(pltpu.async_copy(c2_hbm.at[pl.ds(yt, 128)], cb2, sem))
        for cp in cps:
            cp.wait()
        lane = lax.iota(jnp.int32, _L)
        lx = xs - xt
        ly = ys - yt
        bx = jnp.minimum(lx, 128 - _L)
        by = jnp.minimum(ly, 128 - _L)
        jxv = jnp.full((_L,), lx - bx, jnp.int32)
        jyv = jnp.full((_L,), ly - by, jnp.int32)
        acc = jnp.zeros((_L,), jnp.float32)
        for k in range(8):
            for s in range(8):
                va = abufs[k][s, pl.ds(bx, _L)]
                vb = bbufs[k][s, pl.ds(by, _L)]
                aa = va.at[jxv].get(mode="promise_in_bounds")
                acc = acc + aa * vb
        dot_all = acc.at[jyv].get(mode="promise_in_bounds")
        b1 = cb1[pl.ds(bx, _L)].at[jxv].get(mode="promise_in_bounds")
        b2 = cb2[pl.ds(by, _L)].at[jyv].get(mode="promise_in_bounds")
        res[...] = dot_all + b1 + b2
        pltpu.sync_copy(res.at[pl.ds(0, 1)], out_hbm)


def kernel(x, y, A, B, c1, c2):
    idx = jnp.concatenate([
        jnp.full((8,), x, dtype=jnp.int32),
        jnp.full((8,), y, dtype=jnp.int32),
    ])
    run = pl.kernel(
        _tec_body,
        mesh=plsc.VectorSubcoreMesh(core_axis_name="c", subcore_axis_name="s"),
        out_type=jax.ShapeDtypeStruct((1,), jnp.float32),
        scratch_types=(
            [pltpu.VMEM((_L,), jnp.int32)]
            + [pltpu.VMEM((8, 128), jnp.float32) for _ in range(16)]
            + [pltpu.VMEM((128,), jnp.float32) for _ in range(2)]
            + [pltpu.VMEM((_L,), jnp.float32), pltpu.SemaphoreType.DMA]
        ),
    )
    return run(idx, A.T, B.T, jnp.reshape(c1, (-1,)), jnp.reshape(c2, (-1,)))


# speedup vs baseline: 5.5719x; 1.0698x over previous
import jax
import jax.numpy as jnp
from jax import lax
from jax.experimental import pallas as pl
from jax.experimental.pallas import tpu as pltpu
from jax.experimental.pallas import tpu_sc as plsc

_L = 16
_DIM = 64
_NROW = 100000


def _tec_body(idx_hbm, at_hbm, bt_hbm, c1_hbm, c2_hbm, out_hbm,
              ixv, ab0, ab1, ab2, ab3, ab4, ab5, ab6, ab7,
              bb0, bb1, bb2, bb3, bb4, bb5, bb6, bb7,
              cb1, cb2, res, sem):
    abufs = [ab0, ab1, ab2, ab3, ab4, ab5, ab6, ab7]
    bbufs = [bb0, bb1, bb2, bb3, bb4, bb5, bb6, bb7]
    cid = lax.axis_index("c")
    sid = lax.axis_index("s")

    @pl.when(jnp.logical_and(cid == 0, sid == 0))
    def _():
        pltpu.sync_copy(idx_hbm, ixv)
        iv = ixv[...]
        xs = iv[0]
        ys = iv[8]
        xt = pl.multiple_of((xs // 128) * 128, 128)
        yt = pl.multiple_of((ys // 128) * 128, 128)
        cps = []
        for k in range(8):
            cps.append(pltpu.async_copy(
                at_hbm.at[pl.ds(8 * k, 8), pl.ds(xt, 128)], abufs[k], sem))
            cps.append(pltpu.async_copy(
                bt_hbm.at[pl.ds(8 * k, 8), pl.ds(yt, 128)], bbufs[k], sem))
        cps.append(pltpu.async_copy(c1_hbm.at[pl.ds(xt, 128)], cb1, sem))
        cps.append(pltpu.async_copy(c2_hbm.at[pl.ds(yt, 128)], cb2, sem))
        for cp in cps:
            cp.wait()
        lane = lax.iota(jnp.int32, _L)
        lx = xs - xt
        ly = ys - yt
        bx = jnp.minimum(lx, 128 - _L)
        by = jnp.minimum(ly, 128 - _L)
        jxv = jnp.full((_L,), lx - bx, jnp.int32)
        jyv = jnp.full((_L,), ly - by, jnp.int32)
        acc = jnp.zeros((_L,), jnp.float32)
        for k in range(8):
            for s in range(8):
                va = abufs[k][s, pl.ds(bx, _L)]
                vb = bbufs[k][s, pl.ds(by, _L)]
                aa = va.at[jxv].get(mode="promise_in_bounds")
                acc = acc + aa * vb
        dot_all = acc.at[jyv].get(mode="promise_in_bounds")
        b1 = cb1[pl.ds(bx, _L)].at[jxv].get(mode="promise_in_bounds")
        b2 = cb2[pl.ds(by, _L)].at[jyv].get(mode="promise_in_bounds")
        res[...] = dot_all + b1 + b2
        pltpu.sync_copy(res.at[pl.ds(0, 1)], out_hbm)


def kernel(x, y, A, B, c1, c2):
    idx = jnp.concatenate([
        jnp.full((8,), x, dtype=jnp.int32),
        jnp.full((8,), y, dtype=jnp.int32),
    ])
    run = pl.kernel(
        _tec_body,
        mesh=plsc.VectorSubcoreMesh(core_axis_name="c", subcore_axis_name="s",
                                    num_cores=1, num_subcores=1),
        out_type=jax.ShapeDtypeStruct((1,), jnp.float32),
        scratch_types=(
            [pltpu.VMEM((_L,), jnp.int32)]
            + [pltpu.VMEM((8, 128), jnp.float32) for _ in range(16)]
            + [pltpu.VMEM((128,), jnp.float32) for _ in range(2)]
            + [pltpu.VMEM((_L,), jnp.float32), pltpu.SemaphoreType.DMA]
        ),
    )
    return run(idx, A.T, B.T, jnp.reshape(c1, (-1,)), jnp.reshape(c2, (-1,)))
